# baseline (speedup 1.0000x reference)
import functools

import jax
import jax.numpy as jnp
from jax import lax
from jax.experimental import pallas as pl
from jax.experimental.pallas import tpu as pltpu
from jax.experimental.pallas import tpu_sc as plsc

M = 1000000
D = 32
DE = 16
B = 16384

NC = 2
NS = 16
BT = B // NS
IC = 128
NCHUNK = BT // IC

ROWS_PT = M // NS
TS_PT = 62504
TS_LAST = M - (NS - 1) * TS_PT


def _impl(memory, memory_ts, mailbox, mailbox_ts, idx3, val2, mail2, ts2):
    mesh = plsc.VectorSubcoreMesh(core_axis_name="c", subcore_axis_name="s")

    @functools.partial(
        pl.kernel,
        mesh=mesh,
        out_type=(
            jax.ShapeDtypeStruct((M, D), jnp.float32),
            jax.ShapeDtypeStruct((M,), jnp.float32),
            jax.ShapeDtypeStruct((M, D + DE), jnp.float32),
            jax.ShapeDtypeStruct((M,), jnp.float32),
        ),
        scratch_types=[
            pltpu.VMEM((NCHUNK, IC), jnp.int32),
            pltpu.VMEM((BT, D + DE), jnp.float32),
            pltpu.VMEM((BT, D), jnp.float32),
            pltpu.VMEM((BT,), jnp.float32),
            pltpu.SemaphoreType.DMA,
        ],
        compiler_params=pltpu.CompilerParams(use_tc_tiling_on_sc=False),
    )
    def k(mem_h, memts_h, mail_h, mailts_h, idx3_h, val2_h, mail2_h, ts2_h,
          mem_o, memts_o, mail_o, mailts_o,
          idx_v, mail_v, val_v, ts_v, sem):
        c = lax.axis_index("c")
        s = lax.axis_index("s")

        r0 = s * ROWS_PT
        t0 = s * TS_PT

        @pl.when(c == 0)
        def _copy_mem():
            pltpu.sync_copy(mem_h.at[pl.ds(r0, ROWS_PT)],
                            mem_o.at[pl.ds(r0, ROWS_PT)])

            @pl.when(s < NS - 1)
            def _():
                pltpu.sync_copy(memts_h.at[pl.ds(t0, TS_PT)],
                                memts_o.at[pl.ds(t0, TS_PT)])

            @pl.when(s == NS - 1)
            def _():
                pltpu.sync_copy(memts_h.at[pl.ds(t0, TS_LAST)],
                                memts_o.at[pl.ds(t0, TS_LAST)])

        @pl.when(c == 1)
        def _copy_mail():
            pltpu.sync_copy(mail_h.at[pl.ds(r0, ROWS_PT)],
                            mail_o.at[pl.ds(r0, ROWS_PT)])

            @pl.when(s < NS - 1)
            def _():
                pltpu.sync_copy(mailts_h.at[pl.ds(t0, TS_PT)],
                                mailts_o.at[pl.ds(t0, TS_PT)])

            @pl.when(s == NS - 1)
            def _():
                pltpu.sync_copy(mailts_h.at[pl.ds(t0, TS_LAST)],
                                mailts_o.at[pl.ds(t0, TS_LAST)])

        plsc.subcore_barrier()

        b0 = s * BT
        pltpu.sync_copy(idx3_h.at[s], idx_v)
        pltpu.sync_copy(ts2_h.at[pl.ds(b0, BT)], ts_v)

        @pl.when(c == 0)
        def _scat_mem():
            pltpu.sync_copy(val2_h.at[pl.ds(b0, BT)], val_v)
            for j in range(NCHUNK):
                pltpu.async_copy(
                    val_v.at[pl.ds(j * IC, IC)],
                    mem_o.at[idx_v.at[j]], sem).wait()
                pltpu.async_copy(
                    ts_v.at[pl.ds(j * IC, IC)],
                    memts_o.at[idx_v.at[j]], sem).wait()

        @pl.when(c == 1)
        def _scat_mail():
            pltpu.sync_copy(mail2_h.at[pl.ds(b0, BT)], mail_v)
            for j in range(NCHUNK):
                pltpu.async_copy(
                    mail_v.at[pl.ds(j * IC, IC)],
                    mail_o.at[idx_v.at[j]], sem).wait()
                pltpu.async_copy(
                    ts_v.at[pl.ds(j * IC, IC)],
                    mailts_o.at[idx_v.at[j]], sem).wait()

    return k(memory, memory_ts, mailbox, mailbox_ts, idx3, val2, mail2, ts2)


def kernel(memory, memory_ts, mailbox, mailbox_ts, idx, val, ts, edge_feats):
    iota = jnp.arange(B, dtype=jnp.int32)
    pos = jnp.full((M,), -1, dtype=jnp.int32).at[idx].max(iota)
    win = pos[idx]
    val2 = val[win]
    mail2 = jnp.concatenate([val2, edge_feats[win]], axis=1)
    ts2 = ts[win]
    idx3 = idx.reshape(NS, NCHUNK, IC)
    return _impl(memory, memory_ts, mailbox, mailbox_ts, idx3, val2, mail2, ts2)


# speedup vs baseline: 4.4937x; 4.4937x over previous
import functools

import jax
import jax.numpy as jnp
from jax import lax
from jax.experimental import pallas as pl
from jax.experimental.pallas import tpu as pltpu
from jax.experimental.pallas import tpu_sc as plsc

M = 1000000
D = 32
DE = 16
B = 16384

NC = 2
NS = 16
BT = B // NS
IC = 128
NCHUNK = BT // IC

ROWS_PT = M // NS
TS_PT = 62504
TS_LAST = M - (NS - 1) * TS_PT


def _impl(memory, memory_ts, mailbox, mailbox_ts, idx3, val2, mail2, ts2):
    mesh = plsc.VectorSubcoreMesh(core_axis_name="c", subcore_axis_name="s")

    @functools.partial(
        pl.kernel,
        mesh=mesh,
        out_type=(
            jax.ShapeDtypeStruct((M, D), jnp.float32),
            jax.ShapeDtypeStruct((M,), jnp.float32),
            jax.ShapeDtypeStruct((M, D + DE), jnp.float32),
            jax.ShapeDtypeStruct((M,), jnp.float32),
        ),
        scratch_types=[
            pltpu.VMEM((NCHUNK, IC), jnp.int32),
            pltpu.VMEM((BT, D + DE), jnp.float32),
            pltpu.VMEM((BT, D), jnp.float32),
            pltpu.VMEM((BT,), jnp.float32),
            pltpu.VMEM((2 * 8192,), jnp.float32),
            pltpu.SemaphoreType.DMA,
            pltpu.SemaphoreType.DMA,
            pltpu.SemaphoreType.DMA,
        ],
        compiler_params=pltpu.CompilerParams(use_tc_tiling_on_sc=False),
    )
    def k(mem_h, memts_h, mail_h, mailts_h, idx3_h, val2_h, mail2_h, ts2_h,
          mem_o, memts_o, mail_o, mailts_o,
          idx_v, mail_v, val_v, ts_v, tsbuf_v, sem, in_sem, out_sem):
        c = lax.axis_index("c")
        s = lax.axis_index("s")

        def staged_copy(src, dst, buf_ref, bufrows, base, nrows):
            nfull = nrows // bufrows
            ins = []
            outs = []
            for i in range(nfull):
                b = base + i * bufrows
                half = buf_ref.at[pl.ds((i % 2) * bufrows, bufrows)]
                if i >= 2:
                    outs[i - 2].wait()
                ins.append(pltpu.async_copy(
                    src.at[pl.ds(b, bufrows)], half, in_sem))
                ins[i].wait()
                outs.append(pltpu.async_copy(
                    half, dst.at[pl.ds(b, bufrows)], out_sem))
            rem = nrows - nfull * bufrows
            if rem:
                b = base + nfull * bufrows
                half = buf_ref.at[pl.ds((nfull % 2) * bufrows, rem)]
                if nfull >= 2:
                    outs[nfull - 2].wait()
                pltpu.async_copy(src.at[pl.ds(b, rem)], half, in_sem).wait()
                outs.append(pltpu.async_copy(
                    half, dst.at[pl.ds(b, rem)], out_sem))
            for o in outs[max(0, len(outs) - 2):]:
                o.wait()

        r0 = s * ROWS_PT
        t0 = s * TS_PT

        @pl.when(c == 0)
        def _copy_mem():
            staged_copy(mem_h, mem_o, val_v, BT // 2, r0, ROWS_PT)

            @pl.when(s < NS - 1)
            def _():
                staged_copy(memts_h, memts_o, tsbuf_v, 8192, t0, TS_PT)

            @pl.when(s == NS - 1)
            def _():
                staged_copy(memts_h, memts_o, tsbuf_v, 8192, t0, TS_LAST)

        @pl.when(c == 1)
        def _copy_mail():
            staged_copy(mail_h, mail_o, mail_v, BT // 2, r0, ROWS_PT)

            @pl.when(s < NS - 1)
            def _():
                staged_copy(mailts_h, mailts_o, tsbuf_v, 8192, t0, TS_PT)

            @pl.when(s == NS - 1)
            def _():
                staged_copy(mailts_h, mailts_o, tsbuf_v, 8192, t0, TS_LAST)

        plsc.subcore_barrier()

        b0 = s * BT
        pltpu.sync_copy(idx3_h.at[s], idx_v)
        pltpu.sync_copy(ts2_h.at[pl.ds(b0, BT)], ts_v)

        @pl.when(c == 0)
        def _scat_mem():
            pltpu.sync_copy(val2_h.at[pl.ds(b0, BT)], val_v)
            for j in range(NCHUNK):
                pltpu.async_copy(
                    val_v.at[pl.ds(j * IC, IC)],
                    mem_o.at[idx_v.at[j]], sem).wait()
                pltpu.async_copy(
                    ts_v.at[pl.ds(j * IC, IC)],
                    memts_o.at[idx_v.at[j]], sem).wait()

        @pl.when(c == 1)
        def _scat_mail():
            pltpu.sync_copy(mail2_h.at[pl.ds(b0, BT)], mail_v)
            for j in range(NCHUNK):
                pltpu.async_copy(
                    mail_v.at[pl.ds(j * IC, IC)],
                    mail_o.at[idx_v.at[j]], sem).wait()
                pltpu.async_copy(
                    ts_v.at[pl.ds(j * IC, IC)],
                    mailts_o.at[idx_v.at[j]], sem).wait()

    return k(memory, memory_ts, mailbox, mailbox_ts, idx3, val2, mail2, ts2)


def kernel(memory, memory_ts, mailbox, mailbox_ts, idx, val, ts, edge_feats):
    iota = jnp.arange(B, dtype=jnp.int32)
    pos = jnp.full((M,), -1, dtype=jnp.int32).at[idx].max(iota)
    win = pos[idx]
    val2 = val[win]
    mail2 = jnp.concatenate([val2, edge_feats[win]], axis=1)
    ts2 = ts[win]
    idx3 = idx.reshape(NS, NCHUNK, IC)
    return _impl(memory, memory_ts, mailbox, mailbox_ts, idx3, val2, mail2, ts2)


# speedup vs baseline: 4.9453x; 1.1005x over previous
import functools

import jax
import jax.numpy as jnp
from jax import lax
from jax.experimental import pallas as pl
from jax.experimental.pallas import tpu as pltpu
from jax.experimental.pallas import tpu_sc as plsc

M = 1000000
D = 32
DE = 16
B = 16384

NC = 2
NS = 16
BT = B // NS
IC = 128
NCHUNK = BT // IC


def _scatter_inplace(mem_r, memts_r, mail_r, mailts_r, idx3, val2, mail2, ts2):
    mesh = plsc.VectorSubcoreMesh(core_axis_name="c", subcore_axis_name="s")

    @functools.partial(
        pl.kernel,
        mesh=mesh,
        scratch_types=[
            pltpu.VMEM((NCHUNK, IC), jnp.int32),
            pltpu.VMEM((BT, D + DE), jnp.float32),
            pltpu.VMEM((BT, D), jnp.float32),
            pltpu.VMEM((BT,), jnp.float32),
            pltpu.SemaphoreType.DMA,
        ],
        compiler_params=pltpu.CompilerParams(use_tc_tiling_on_sc=False),
    )
    def k(mem_o, memts_o, mail_o, mailts_o, idx3_h, val2_h, mail2_h, ts2_h,
          idx_v, mail_v, val_v, ts_v, sem):
        c = lax.axis_index("c")
        s = lax.axis_index("s")

        b0 = s * BT
        pltpu.sync_copy(idx3_h.at[s], idx_v)
        pltpu.sync_copy(ts2_h.at[pl.ds(b0, BT)], ts_v)

        @pl.when(c == 0)
        def _scat_mem():
            pltpu.sync_copy(val2_h.at[pl.ds(b0, BT)], val_v)
            for j in range(NCHUNK):
                pltpu.async_copy(
                    val_v.at[pl.ds(j * IC, IC)],
                    mem_o.at[idx_v.at[j]], sem).wait()
                pltpu.async_copy(
                    ts_v.at[pl.ds(j * IC, IC)],
                    memts_o.at[idx_v.at[j]], sem).wait()

        @pl.when(c == 1)
        def _scat_mail():
            pltpu.sync_copy(mail2_h.at[pl.ds(b0, BT)], mail_v)
            for j in range(NCHUNK):
                pltpu.async_copy(
                    mail_v.at[pl.ds(j * IC, IC)],
                    mail_o.at[idx_v.at[j]], sem).wait()
                pltpu.async_copy(
                    ts_v.at[pl.ds(j * IC, IC)],
                    mailts_o.at[idx_v.at[j]], sem).wait()

    return k(mem_r, memts_r, mail_r, mailts_r, idx3, val2, mail2, ts2)


def kernel(memory, memory_ts, mailbox, mailbox_ts, idx, val, ts, edge_feats):
    iota = jnp.arange(B, dtype=jnp.int32)
    pos = jnp.full((M,), -1, dtype=jnp.int32).at[idx].max(iota)
    win = pos[idx]
    val2 = val[win]
    mail2 = jnp.concatenate([val2, edge_feats[win]], axis=1)
    ts2 = ts[win]
    idx3 = idx.reshape(NS, NCHUNK, IC)

    mem_r = jax.new_ref(memory)
    memts_r = jax.new_ref(memory_ts)
    mail_r = jax.new_ref(mailbox)
    mailts_r = jax.new_ref(mailbox_ts)
    _scatter_inplace(mem_r, memts_r, mail_r, mailts_r, idx3, val2, mail2, ts2)
    return mem_r[...], memts_r[...], mail_r[...], mailts_r[...]
